# Initial kernel scaffold; baseline (speedup 1.0000x reference)
import jax
import jax.numpy as jnp
from jax.experimental import pallas as pl


def kernel(logits, sem_gt, is_train, mav_table, prev_count):
    raise NotImplementedError("write your pallas kernel here")


# speedup vs baseline: 3.8898x; 3.8898x over previous
import jax
import jax.numpy as jnp
from jax.experimental import pallas as pl
from jax.experimental.pallas import tpu as pltpu

_C = 19
_B = 8192
_EPS = 1e-8


def _owloss_tc_kernel(g_ref, x_ref, mav_ref, pc_ref, out_ref, sacc, cacc):
    i = pl.program_id(0)
    nsteps = pl.num_programs(0)

    @pl.when(i == 0)
    def _init():
        sacc[...] = jnp.zeros_like(sacc)
        cacc[...] = jnp.zeros_like(cacc)

    hi = jax.lax.Precision.HIGHEST
    x = x_ref[...]
    g = g_ref[0]
    mav = mav_ref[...]

    at = jax.lax.dot_general(mav, x, (((1,), (1,)), ((), ())), precision=hi)
    ones_row = jnp.ones((1, _C), jnp.float32)
    nsq = jax.lax.dot_general(ones_row, x * x, (((1,), (1,)), ((), ())),
                              precision=hi)
    lbl = jax.lax.broadcasted_iota(jnp.int32, (_C, 1), 0)
    oh = (lbl == g).astype(jnp.float32)
    num = jax.lax.dot_general(ones_row, oh * at, (((1,), (0,)), ((), ())),
                              precision=hi)
    mavn = jnp.sqrt(jnp.sum(mav * mav, axis=1, keepdims=True))
    nm = jax.lax.dot_general(mavn, oh, (((0,), (0,)), ((), ())), precision=hi)
    den = jnp.maximum(jnp.sqrt(nsq) * nm, _EPS)
    ew = 1.0 - num / den

    sacc[...] += oh * ew
    cacc[...] += oh

    @pl.when(i == nsteps - 1)
    def _finish():
        s = jnp.sum(sacc[...], axis=1, keepdims=True)
        c = jnp.sum(cacc[...], axis=1, keepdims=True)
        pc = pc_ref[...]
        present = c > 0.0
        minl = jnp.min(jnp.where(present, lbl, _C))
        include = present & (lbl != minl) & (pc > 0.0)
        terms = jnp.where(include, s / jnp.maximum(c, 1.0), 0.0)
        out_ref[...] = jnp.sum(terms, axis=(0, 1), keepdims=True).reshape(1, 1)


def kernel(logits, sem_gt, is_train, mav_table, prev_count):
    n = logits.shape[0]
    nsteps = n // _B
    g3 = sem_gt.reshape(nsteps, 1, _B)
    pc2 = prev_count.reshape(_C, 1)
    out = pl.pallas_call(
        _owloss_tc_kernel,
        grid=(nsteps,),
        in_specs=[
            pl.BlockSpec((1, 1, _B), lambda i: (i, 0, 0)),
            pl.BlockSpec((_B, _C), lambda i: (i, 0)),
            pl.BlockSpec((_C, _C), lambda i: (0, 0)),
            pl.BlockSpec((_C, 1), lambda i: (0, 0)),
        ],
        out_specs=pl.BlockSpec((1, 1), lambda i: (0, 0)),
        out_shape=jax.ShapeDtypeStruct((1, 1), jnp.float32),
        scratch_shapes=[
            pltpu.VMEM((_C, _B), jnp.float32),
            pltpu.VMEM((_C, _B), jnp.float32),
        ],
        compiler_params=pltpu.CompilerParams(
            dimension_semantics=("arbitrary",),
        ),
    )(g3, logits, mav_table, pc2)
    return jnp.reshape(out, ())


# speedup vs baseline: 6.2874x; 1.6164x over previous
import jax
import jax.numpy as jnp
from jax.experimental import pallas as pl
from jax.experimental.pallas import tpu as pltpu

_C = 19
_B = 16384
_EPS = 1e-30


def _owloss_tc_kernel(g_ref, x_ref, mav_ref, pc_ref, out_ref, sacc, cacc):
    i = pl.program_id(0)
    nsteps = pl.num_programs(0)

    @pl.when(i == 0)
    def _init():
        sacc[...] = jnp.zeros_like(sacc)
        cacc[...] = jnp.zeros_like(cacc)

    hi = jax.lax.Precision.HIGHEST
    x = x_ref[...]
    g = g_ref[0]
    mav = mav_ref[...]

    mns = jnp.sum(mav * mav, axis=1, keepdims=True)
    w = mav * jax.lax.rsqrt(jnp.maximum(mns, _EPS))
    at = jax.lax.dot_general(w, x, (((1,), (1,)), ((), ())), precision=hi)
    xb = x.astype(jnp.bfloat16)
    ones_row = jnp.ones((1, _C), jnp.bfloat16)
    nsq = jax.lax.dot_general(ones_row, xb * xb, (((1,), (1,)), ((), ())),
                              preferred_element_type=jnp.float32)
    rnl = jax.lax.rsqrt(jnp.maximum(nsq, _EPS))

    lbl = jax.lax.broadcasted_iota(jnp.int32, (_C, 1), 0)
    oh = (lbl == g).astype(jnp.float32)
    num = jnp.sum(oh * at, axis=0, keepdims=True)
    ew = 1.0 - num * rnl

    sacc[...] += oh * ew
    cacc[...] += oh

    @pl.when(i == nsteps - 1)
    def _finish():
        s = jnp.sum(sacc[...], axis=1, keepdims=True)
        c = jnp.sum(cacc[...], axis=1, keepdims=True)
        pc = pc_ref[...]
        present = c > 0.0
        minl = jnp.min(jnp.where(present, lbl, _C))
        include = present & (lbl != minl) & (pc > 0.0)
        terms = jnp.where(include, s / jnp.maximum(c, 1.0), 0.0)
        out_ref[...] = jnp.sum(terms, axis=(0, 1), keepdims=True).reshape(1, 1)


def kernel(logits, sem_gt, is_train, mav_table, prev_count):
    n = logits.shape[0]
    nsteps = n // _B
    g3 = sem_gt.reshape(nsteps, 1, _B)
    pc2 = prev_count.reshape(_C, 1)
    out = pl.pallas_call(
        _owloss_tc_kernel,
        grid=(nsteps,),
        in_specs=[
            pl.BlockSpec((1, 1, _B), lambda i: (i, 0, 0)),
            pl.BlockSpec((_B, _C), lambda i: (i, 0)),
            pl.BlockSpec((_C, _C), lambda i: (0, 0)),
            pl.BlockSpec((_C, 1), lambda i: (0, 0)),
        ],
        out_specs=pl.BlockSpec((1, 1), lambda i: (0, 0)),
        out_shape=jax.ShapeDtypeStruct((1, 1), jnp.float32),
        scratch_shapes=[
            pltpu.VMEM((_C, _B), jnp.float32),
            pltpu.VMEM((_C, _B), jnp.float32),
        ],
        compiler_params=pltpu.CompilerParams(
            dimension_semantics=("arbitrary",),
        ),
    )(g3, logits, mav_table, pc2)
    return jnp.reshape(out, ())


# speedup vs baseline: 10.1945x; 1.6214x over previous
import jax
import jax.numpy as jnp
from jax.experimental import pallas as pl
from jax.experimental.pallas import tpu as pltpu

_C = 19
_B = 16384
_EPS = 1e-30


def _owloss_tc_kernel(g_ref, x_ref, mav_ref, pc_ref, out_ref, wb_ref, acc):
    i = pl.program_id(0)
    nsteps = pl.num_programs(0)

    @pl.when(i == 0)
    def _init():
        acc[...] = jnp.zeros_like(acc)
        mav = mav_ref[...]
        mns = jnp.sum(mav * mav, axis=1, keepdims=True)
        w = mav * jax.lax.rsqrt(jnp.maximum(mns, _EPS))
        wb_ref[...] = w.astype(jnp.bfloat16)

    x = x_ref[...]
    g = g_ref[0]
    xb = x.astype(jnp.bfloat16)

    at = jax.lax.dot_general(wb_ref[...], xb, (((1,), (1,)), ((), ())),
                             preferred_element_type=jnp.float32)
    ones_row = jnp.ones((1, _C), jnp.bfloat16)
    nsq = jax.lax.dot_general(ones_row, xb * xb, (((1,), (1,)), ((), ())),
                              preferred_element_type=jnp.float32)
    rnl = jax.lax.rsqrt(jnp.maximum(nsq, _EPS))

    lbl = jax.lax.broadcasted_iota(jnp.int32, (_C, 1), 0)
    msk = lbl == g
    num = jnp.sum(jnp.where(msk, at, 0.0), axis=0, keepdims=True)
    cos = (num * rnl).astype(jnp.bfloat16)
    ohb = msk.astype(jnp.bfloat16)
    cat = jnp.concatenate([cos, jnp.ones((1, _B), jnp.bfloat16)], axis=0)
    z = jax.lax.dot_general(ohb, cat, (((1,), (1,)), ((), ())),
                            preferred_element_type=jnp.float32)
    acc[...] += z

    @pl.when(i == nsteps - 1)
    def _finish():
        cs = acc[:, 0:1]
        c = acc[:, 1:2]
        pc = pc_ref[...]
        present = c > 0.0
        minl = jnp.min(jnp.where(present, lbl, _C))
        include = present & (lbl != minl) & (pc > 0.0)
        means = (c - cs) / jnp.maximum(c, 1.0)
        terms = jnp.where(include, means, 0.0)
        out_ref[...] = jnp.sum(terms, axis=(0, 1), keepdims=True).reshape(1, 1)


def kernel(logits, sem_gt, is_train, mav_table, prev_count):
    n = logits.shape[0]
    nsteps = n // _B
    g3 = sem_gt.reshape(nsteps, 1, _B)
    pc2 = prev_count.reshape(_C, 1)
    out = pl.pallas_call(
        _owloss_tc_kernel,
        grid=(nsteps,),
        in_specs=[
            pl.BlockSpec((1, 1, _B), lambda i: (i, 0, 0)),
            pl.BlockSpec((_B, _C), lambda i: (i, 0)),
            pl.BlockSpec((_C, _C), lambda i: (0, 0)),
            pl.BlockSpec((_C, 1), lambda i: (0, 0)),
        ],
        out_specs=pl.BlockSpec((1, 1), lambda i: (0, 0)),
        out_shape=jax.ShapeDtypeStruct((1, 1), jnp.float32),
        scratch_shapes=[
            pltpu.VMEM((_C, _C), jnp.bfloat16),
            pltpu.VMEM((_C, 2), jnp.float32),
        ],
        compiler_params=pltpu.CompilerParams(
            dimension_semantics=("arbitrary",),
        ),
    )(g3, logits, mav_table, pc2)
    return jnp.reshape(out, ())
